# Initial kernel scaffold; baseline (speedup 1.0000x reference)
import jax
import jax.numpy as jnp
from jax.experimental import pallas as pl


def kernel(e_index, batch, coarse_h_prob, e, z, timestep, frag_zs, W_lz, b_lz, W_ts, b_ts, W_ce, b_ce, Wg1, bg1, Wg2, bg2, Wm1, bm1, Wm2, bm2, Wn1, bn1, Wn2, bn2, Wgl, bgl, We1, be1, We2, be2, Wz1, bz1, Wz2, bz2):
    raise NotImplementedError("write your pallas kernel here")


# speedup vs baseline: 1.8186x; 1.8186x over previous
import functools

import jax
import jax.numpy as jnp
from jax import lax
from jax.experimental import pallas as pl
from jax.experimental.pallas import tpu as pltpu
from jax.experimental.pallas import tpu_sc as plsc

N = 10000
E = 160000
E2 = 2 * E
NF = 512
BS = 64
HD = 128
ED = 64
ZD = 64
HID = 128

NW = 32
CH = 128
PER_W = 10240
EP = NW * PER_W
NCH = PER_W // CH
NS_ROWS = 10240
TRASH = N

@functools.lru_cache(maxsize=None)
def _sc_mesh():
    return plsc.VectorSubcoreMesh(core_axis_name="c", subcore_axis_name="s")


def _dot(a, b):
    return jax.lax.dot_general(a, b, (((a.ndim - 1,), (0,)), ((), ())),
                               preferred_element_type=jnp.float32,
                               precision=jax.lax.Precision.HIGHEST)


def _sc_gather_body(S_hbm, T_hbm, src_hbm, dst_hbm, GS_hbm, GT_hbm,
                    idxs_v, idxd_v, bufS0, bufS1, bufT0, bufT1,
                    sem0, sem1):
    bufS = (bufS0, bufS1)
    bufT = (bufT0, bufT1)
    sems = (sem0, sem1)
    wid = lax.axis_index("s") * 2 + lax.axis_index("c")
    base = wid * PER_W
    pltpu.sync_copy(src_hbm.at[wid], idxs_v)
    pltpu.sync_copy(dst_hbm.at[wid], idxd_v)

    def pair_body(p, carry):
        descs = []
        for b in range(2):
            c = 2 * p + b
            dS = pltpu.async_copy(
                S_hbm.at[idxs_v.at[c]], bufS[b], sems[b])
            dT = pltpu.async_copy(
                T_hbm.at[idxd_v.at[c]], bufT[b], sems[b])
            descs.append((c, dS, dT))
        for b in range(2):
            c, dS, dT = descs[b]
            dS.wait()
            dT.wait()
            pltpu.sync_copy(bufS[b], GS_hbm.at[pl.ds(base + c * CH, CH)])
            pltpu.sync_copy(bufT[b], GT_hbm.at[pl.ds(base + c * CH, CH)])
        return carry

    lax.fori_loop(0, NCH // 2, pair_body, 0)


@functools.lru_cache(maxsize=None)
def _sc_gather_call():
    return pl.kernel(
        _sc_gather_body,
        out_type=(jax.ShapeDtypeStruct((EP, HD), jnp.float32),
                  jax.ShapeDtypeStruct((EP, HD), jnp.float32)),
        mesh=_sc_mesh(),
        scratch_types=[
            pltpu.VMEM((NCH, CH), jnp.int32),
            pltpu.VMEM((NCH, CH), jnp.int32),
            pltpu.VMEM((CH, HD), jnp.float32),
            pltpu.VMEM((CH, HD), jnp.float32),
            pltpu.VMEM((CH, HD), jnp.float32),
            pltpu.VMEM((CH, HD), jnp.float32),
            pltpu.SemaphoreType.DMA,
            pltpu.SemaphoreType.DMA,
        ],
    )


def _sc_gather(S, T, src, dst):
    return _sc_gather_call()(S, T, src, dst)


def _sc_scatter_body(vals_hbm, idx2_hbm, zeros_hbm, out_hbm,
                     shared, idxA, idxB, val0, val1,
                     sem0, sem1, isem0, isem1):
    vals = (val0, val1)
    idxs = (idxA, idxB)
    sems = (sem0, sem1)
    isems = (isem0, isem1)
    cid = lax.axis_index("c")
    sid = lax.axis_index("s")
    slab = cid * 16 + sid
    base = slab * PER_W
    rows_per_sub = NS_ROWS // 16

    pltpu.sync_copy(zeros_hbm.at[pl.ds(sid * rows_per_sub, rows_per_sub)],
                    shared.at[pl.ds(sid * rows_per_sub, rows_per_sub)])
    plsc.subcore_barrier()

    def pair_body(p, carry):
        descs = []
        for b in range(2):
            c = 2 * p + b
            dv = pltpu.async_copy(
                vals_hbm.at[pl.ds(base + c * CH, CH)], vals[b], sems[b])
            di = pltpu.async_copy(
                idx2_hbm.at[slab * NCH + c], idxs[b], isems[b])
            descs.append((dv, di))
        for b in range(2):
            dv, di = descs[b]
            di.wait()
            dv.wait()
            pltpu.sync_copy(vals[b], shared.at[idxs[b]], add=True)
        return carry

    lax.fori_loop(0, NCH // 2, pair_body, 0)
    plsc.subcore_barrier()
    pltpu.sync_copy(
        shared.at[pl.ds(sid * rows_per_sub, rows_per_sub)],
        out_hbm.at[pl.ds(cid * NS_ROWS + sid * rows_per_sub, rows_per_sub)])


@functools.lru_cache(maxsize=None)
def _sc_scatter_call():
    return pl.kernel(
        _sc_scatter_body,
        out_type=jax.ShapeDtypeStruct((2 * NS_ROWS, HD), jnp.float32),
        mesh=_sc_mesh(),
        scratch_types=[
            pltpu.VMEM_SHARED((NS_ROWS, HD), jnp.float32),
            pltpu.VMEM((CH,), jnp.int32),
            pltpu.VMEM((CH,), jnp.int32),
            pltpu.VMEM((CH, HD), jnp.float32),
            pltpu.VMEM((CH, HD), jnp.float32),
            pltpu.SemaphoreType.DMA,
            pltpu.SemaphoreType.DMA,
            pltpu.SemaphoreType.DMA,
            pltpu.SemaphoreType.DMA,
        ],
    )


def _sc_scatter(sv, idx2, zeros):
    flat = _sc_scatter_call()(sv, idx2, zeros)
    return flat.reshape(2, NS_ROWS, HD)


NB = 1000
NGRID = N // NB


def _h_embed_kernel(p_ref, fz_ref, out_ref):
    out_ref[...] = _dot(p_ref[...], fz_ref[...])


def _h_embed(p, fz):
    return pl.pallas_call(
        _h_embed_kernel,
        grid=(NGRID,),
        in_specs=[pl.BlockSpec((NB, NF), lambda i: (i, 0)),
                  pl.BlockSpec((NF, HD), lambda i: (0, 0))],
        out_specs=pl.BlockSpec((NB, HD), lambda i: (i, 0)),
        out_shape=jax.ShapeDtypeStruct((N, HD), jnp.float32),
    )(p, fz)


EB_EMB = 2000


def _e_emb_kernel(e_ref, w_ref, b_ref, out_ref):
    e = e_ref[...]
    w = w_ref[...]
    out_ref[...] = e[:, 0:1] * w[0:1, :] + e[:, 1:2] * w[1:2, :] + b_ref[...]


def _e_emb(e, W_ce, b_ce):
    return pl.pallas_call(
        _e_emb_kernel,
        grid=(E // EB_EMB,),
        in_specs=[pl.BlockSpec((EB_EMB, 2), lambda i: (i, 0)),
                  pl.BlockSpec((2, ED), lambda i: (0, 0)),
                  pl.BlockSpec((1, ED), lambda i: (0, 0))],
        out_specs=pl.BlockSpec((EB_EMB, ED), lambda i: (i, 0)),
        out_shape=jax.ShapeDtypeStruct((E, ED), jnp.float32),
    )(e, W_ce, b_ce.reshape(1, ED))


def _node_terms_kernel(h_ref, wa_ref, wb_ref, gw_ref, batch_ref, bm_ref,
                       s_ref, t_ref):
    h = h_ref[...]
    b = batch_ref[0, 0, :]
    oh = (b[:, None] == lax.broadcasted_iota(jnp.int32, (NB, BS), 1)
          ).astype(jnp.float32)
    s_ref[...] = _dot(h, wa_ref[...]) + _dot(oh, gw_ref[...]) + bm_ref[...]
    t_ref[...] = _dot(h, wb_ref[...])


def _node_terms(h, Wa, Wb, gWd, batch3, bm1):
    return pl.pallas_call(
        _node_terms_kernel,
        grid=(NGRID,),
        in_specs=[pl.BlockSpec((NB, HD), lambda i: (i, 0)),
                  pl.BlockSpec((HD, HD), lambda i: (0, 0)),
                  pl.BlockSpec((HD, HD), lambda i: (0, 0)),
                  pl.BlockSpec((BS, HD), lambda i: (0, 0)),
                  pl.BlockSpec((1, 1, NB), lambda i: (i, 0, 0)),
                  pl.BlockSpec((1, HD), lambda i: (0, 0))],
        out_specs=[pl.BlockSpec((NB, HD), lambda i: (i, 0)),
                   pl.BlockSpec((NB, HD), lambda i: (i, 0))],
        out_shape=[jax.ShapeDtypeStruct((N, HD), jnp.float32),
                   jax.ShapeDtypeStruct((N, HD), jnp.float32)],
    )(h, Wa, Wb, gWd, batch3, bm1.reshape(1, HD))


EB = 2048


def _edge_mlp_kernel(gs_ref, gt_ref, eb_ref, wc_ref, wm2_ref, bm2_ref, wnb_ref,
                     out_ref, sv_ref):
    eb = eb_ref[...]
    pre = gs_ref[...] + gt_ref[...] + _dot(eb, wc_ref[...])
    m = _dot(jnp.maximum(pre, 0.0), wm2_ref[...]) + bm2_ref[...]
    e_out = eb + m
    out_ref[...] = e_out
    sv_ref[...] = _dot(e_out, wnb_ref[...])


def _edge_mlp(GS, GT, ebidi, Wc, Wm2, bm2, Wnb):
    return pl.pallas_call(
        _edge_mlp_kernel,
        grid=(EP // EB,),
        in_specs=[pl.BlockSpec((EB, HD), lambda i: (i, 0)),
                  pl.BlockSpec((EB, HD), lambda i: (i, 0)),
                  pl.BlockSpec((EB, ED), lambda i: (i, 0)),
                  pl.BlockSpec((ED, HD), lambda i: (0, 0)),
                  pl.BlockSpec((HID, ED), lambda i: (0, 0)),
                  pl.BlockSpec((1, ED), lambda i: (0, 0)),
                  pl.BlockSpec((ED, HD), lambda i: (0, 0))],
        out_specs=[pl.BlockSpec((EB, ED), lambda i: (i, 0)),
                   pl.BlockSpec((EB, HD), lambda i: (i, 0))],
        out_shape=[jax.ShapeDtypeStruct((EP, ED), jnp.float32),
                   jax.ShapeDtypeStruct((EP, HD), jnp.float32)],
    )(GS, GT, ebidi, Wc, Wm2, bm2.reshape(1, ED), Wnb)


def _node_update_kernel(h_ref, agg_ref, batch_ref, wna_ref,
                        gw_ref, bn1_ref, wn2_ref, bn2_ref,
                        h_out, hg_out, cnt_out):
    h = h_ref[...]
    aggw = agg_ref[0] + agg_ref[1]
    b = batch_ref[0, 0, :]
    oh = (b[:, None] == lax.broadcasted_iota(jnp.int32, (NB, BS), 1)
          ).astype(jnp.float32)
    pre = (_dot(h, wna_ref[...]) + aggw
           + _dot(oh, gw_ref[...]) + bn1_ref[...])
    hn = h + _dot(jnp.maximum(pre, 0.0), wn2_ref[...]) + bn2_ref[...]
    h_out[...] = hn

    @pl.when(pl.program_id(0) == 0)
    def _():
        hg_out[...] = jnp.zeros_like(hg_out)
        cnt_out[...] = jnp.zeros_like(cnt_out)

    pool = jax.lax.dot_general(oh, hn, (((0,), (0,)), ((), ())),
                               preferred_element_type=jnp.float32)
    hg_out[...] += pool
    cnt = jnp.sum(oh, axis=0)
    cnt_out[...] += jnp.broadcast_to(cnt[:, None], (BS, HD))


def _node_update(h, partials, batch3, Wna, gWnc, bn1, Wn2, bn2):
    return pl.pallas_call(
        _node_update_kernel,
        grid=(NGRID,),
        in_specs=[pl.BlockSpec((NB, HD), lambda i: (i, 0)),
                  pl.BlockSpec((2, NB, HD), lambda i: (0, i, 0)),
                  pl.BlockSpec((1, 1, NB), lambda i: (i, 0, 0)),
                  pl.BlockSpec((HD, HD), lambda i: (0, 0)),
                  pl.BlockSpec((BS, HD), lambda i: (0, 0)),
                  pl.BlockSpec((1, HD), lambda i: (0, 0)),
                  pl.BlockSpec((HD, HD), lambda i: (0, 0)),
                  pl.BlockSpec((1, HD), lambda i: (0, 0))],
        out_specs=[pl.BlockSpec((NB, HD), lambda i: (i, 0)),
                   pl.BlockSpec((BS, HD), lambda i: (0, 0)),
                   pl.BlockSpec((BS, HD), lambda i: (0, 0))],
        out_shape=[jax.ShapeDtypeStruct((N, HD), jnp.float32),
                   jax.ShapeDtypeStruct((BS, HD), jnp.float32),
                   jax.ShapeDtypeStruct((BS, HD), jnp.float32)],
    )(h, partials, batch3, Wna, gWnc, bn1.reshape(1, HD),
      Wn2, bn2.reshape(1, HD))


EHB = 2000


def _edge_head_kernel(e_ref, w1_ref, b1_ref, w2_ref, b2_ref, out_ref):
    x = jnp.maximum(_dot(e_ref[...], w1_ref[...]) + b1_ref[...], 0.0)
    out_ref[...] = _dot(x, w2_ref[...]) + b2_ref[...]


def _edge_head(e_half, We1, be1, We2p, be2p):
    return pl.pallas_call(
        _edge_head_kernel,
        grid=(E // EHB,),
        in_specs=[pl.BlockSpec((EHB, ED), lambda i: (i, 0)),
                  pl.BlockSpec((ED, ED), lambda i: (0, 0)),
                  pl.BlockSpec((1, ED), lambda i: (0, 0)),
                  pl.BlockSpec((ED, HD), lambda i: (0, 0)),
                  pl.BlockSpec((1, HD), lambda i: (0, 0))],
        out_specs=pl.BlockSpec((EHB, HD), lambda i: (i, 0)),
        out_shape=jax.ShapeDtypeStruct((E, HD), jnp.float32),
    )(e_half, We1, be1.reshape(1, ED), We2p, be2p)


def kernel(e_index, batch, coarse_h_prob, e, z, timestep, frag_zs, W_lz, b_lz,
           W_ts, b_ts, W_ce, b_ce, Wg1, bg1, Wg2, bg2, Wm1, bm1, Wm2, bm2,
           Wn1, bn1, Wn2, bn2, Wgl, bgl, We1, be1, We2, be2, Wz1, bz1,
           Wz2, bz2):
    relu = lambda x: jnp.maximum(x, 0.0)
    e_index = e_index.astype(jnp.int32)
    batch = batch.astype(jnp.int32)

    src_half = e_index[0]
    dst_half = e_index[1]
    pad_i = jnp.zeros((EP - E2,), jnp.int32)
    src_all = jnp.concatenate([src_half, dst_half, pad_i]).reshape(NW, NCH, CH)
    dst_all = jnp.concatenate([dst_half, src_half, pad_i]).reshape(NW, NCH, CH)
    dst_scatter = jnp.concatenate(
        [dst_half, src_half, jnp.full((EP - E2,), TRASH, jnp.int32)]
    ).reshape(NW * NCH, CH)
    batch3 = batch.reshape(NGRID, 1, NB)
    zeros_tbl = jnp.zeros((NS_ROWS, HD), jnp.float32)

    h = _h_embed(coarse_h_prob, frag_zs)
    e_emb = _e_emb(e, W_ce, b_ce)
    ebidi = jnp.concatenate(
        [e_emb, e_emb, jnp.zeros((EP - E2, ED), jnp.float32)], axis=0)

    frag_bag = jnp.mean(frag_zs, axis=0)
    ts = timestep[:, None] @ W_ts + b_ts
    lz = z @ W_lz + b_lz
    fb = jnp.broadcast_to(frag_bag[None, :], (BS, HD))
    gin = jnp.concatenate([fb, ts, lz], axis=1)
    g = relu(gin @ Wg1 + bg1) @ Wg2 + bg2

    for l in range(Wm1.shape[0]):
        Wa = Wm1[l][:HD]
        Wb = Wm1[l][HD:2 * HD]
        Wc = Wm1[l][2 * HD:2 * HD + ED]
        Wd = Wm1[l][2 * HD + ED:]
        gWd = g @ Wd
        Wna = Wn1[l][:HD]
        Wnb = Wn1[l][HD:HD + ED]
        Wnc = Wn1[l][HD + ED:]
        S, T = _node_terms(h, Wa, Wb, gWd, batch3, bm1[l])
        GS, GT = _sc_gather(S, T, src_all, dst_all)
        ebidi, sv = _edge_mlp(GS, GT, ebidi, Wc, Wm2[l], bm2[l], Wnb)
        partials = _sc_scatter(sv, dst_scatter, zeros_tbl)
        gWnc = g @ Wnc
        h, hg_sum, cnt_b = _node_update(
            h, partials, batch3, Wna, gWnc, bn1[l], Wn2[l], bn2[l])
        cnt = cnt_b[:, 0]
        hg = hg_sum / jnp.clip(cnt, 1.0)[:, None]
        g = g + jnp.concatenate([hg, g], axis=1) @ Wgl[l] + bgl[l]

    We2p = jnp.zeros((ED, HD), jnp.float32).at[:, :2].set(We2)
    be2p = jnp.zeros((1, HD), jnp.float32).at[0, :2].set(be2)
    e_logit = _edge_head(ebidi[:E], We1, be1, We2p, be2p)[:, :2]
    z_out = relu(g @ Wz1 + bz1) @ Wz2 + bz2
    return (h, e_logit, z_out)
